# baseline (speedup 1.0000x reference)
import functools

import jax
import jax.numpy as jnp
from jax import lax
from jax.experimental import pallas as pl
from jax.experimental.pallas import tpu as pltpu
from jax.experimental.pallas import tpu_sc as plsc

B = 1048576
RES = 512
C = 6
VFAIL = 6 * RES * RES

NC, NS = 2, 16
NW = NC * NS
RW = B // NW
CB = 1024
NCH = RW // CB
GB = 128
NG = CB // GB


def _phase_a(g, xyzv, wsv, wtv, i0, i1, i2, i3):
    off = g * 16
    iota = lax.iota(jnp.int32, 16)
    i3v = iota * 3 + off * 3
    x = plsc.load_gather(xyzv, [i3v])
    y = plsc.load_gather(xyzv, [i3v + 1])
    z = plsc.load_gather(xyzv, [i3v + 2])
    ax, ay, az = jnp.abs(x), jnp.abs(y), jnp.abs(z)
    is_x = (ax >= ay) & (ax >= az)
    is_y = (~is_x) & (ay >= az)
    ma = jnp.maximum(jnp.maximum(ax, ay), az)
    ms = jnp.maximum(ma, 1e-12)
    xpos, ypos, zpos = x >= 0, y >= 0, z >= 0
    u = jnp.where(is_x, jnp.where(xpos, -z, z),
                  jnp.where(is_y, x, jnp.where(zpos, x, -x))) / ms
    v = jnp.where(is_x, -y, jnp.where(is_y, jnp.where(ypos, z, -z), -y)) / ms
    face = jnp.where(is_x, jnp.where(xpos, 0, 1),
                     jnp.where(is_y, jnp.where(ypos, 2, 3),
                               jnp.where(zpos, 4, 5))).astype(jnp.int32)
    s = (u * 0.5 + 0.5) * RES - 0.5
    t = (v * 0.5 + 0.5) * RES - 0.5

    def flr(s):
        si_t = s.astype(jnp.int32)
        sf_t = si_t.astype(jnp.float32)
        adj = sf_t > s
        si = jnp.where(adj, si_t - 1, si_t)
        sf = jnp.where(adj, sf_t - 1.0, sf_t)
        return si, s - sf

    s0i, ws = flr(s)
    t0i, wt = flr(t)
    s0 = jnp.clip(s0i, 0, RES - 1)
    s1 = jnp.clip(s0i + 1, 0, RES - 1)
    t0 = jnp.clip(t0i, 0, RES - 1)
    t1 = jnp.clip(t0i + 1, 0, RES - 1)
    rb = face << 18
    r0 = rb + (t0 << 9)
    r1 = rb + (t1 << 9)
    fail = ma < 1e-12
    idx00 = jnp.where(fail, VFAIL, r0 + s0)
    idx01 = jnp.where(fail, VFAIL, r0 + s1)
    idx10 = jnp.where(fail, VFAIL, r1 + s0)
    idx11 = jnp.where(fail, VFAIL, r1 + s1)

    sl = pl.ds(off, 16)
    wsv[sl] = ws
    wtv[sl] = wt
    i0[sl] = idx00
    i1[sl] = idx01
    i2[sl] = idx10
    i3[sl] = idx11


def _phase_b(q, wsv, wtv, t00, t01, t10, t11, outv):
    off = q * 16
    iota = lax.iota(jnp.int32, 16)
    rowv = off + iota
    obase = off * 6 + iota * 6
    sl = pl.ds(off, 16)
    ws = wsv[sl]
    wt = wtv[sl]
    for ch in range(C):
        chv = jnp.full((16,), ch, jnp.int32)
        a00 = plsc.load_gather(t00, [rowv, chv])
        a01 = plsc.load_gather(t01, [rowv, chv])
        a10 = plsc.load_gather(t10, [rowv, chv])
        a11 = plsc.load_gather(t11, [rowv, chv])
        a = a00 + ws * (a01 - a00)
        b = a10 + ws * (a11 - a10)
        o = a + wt * (b - a)
        plsc.store_scatter(outv, [obase + ch], o)


def _sc_lookup(xyz_hbm, table_hbm, out_hbm,
               xyzv, wsv, wtv, i0, i1, i2, i3,
               t00, t01, t10, t11, outv, sem):
    wid = lax.axis_index("s") * NC + lax.axis_index("c")
    base0 = wid * RW

    def chunk_body(c, carry):
        base = base0 + c * CB
        pltpu.sync_copy(xyz_hbm.at[pl.ds(base * 3, CB * 3)], xyzv)
        lax.fori_loop(
            0, CB // 16,
            lambda g, _: (_phase_a(g, xyzv, wsv, wtv, i0, i1, i2, i3), 0)[1],
            0)
        handles = []
        for iref, tref in ((i0, t00), (i1, t01), (i2, t10), (i3, t11)):
            for j in range(NG):
                handles.append(pltpu.async_copy(
                    table_hbm.at[iref.at[pl.ds(j * GB, GB)]],
                    tref.at[pl.ds(j * GB, GB)], sem))
        for h in handles:
            h.wait()
        lax.fori_loop(
            0, CB // 16,
            lambda q, _: (_phase_b(q, wsv, wtv, t00, t01, t10, t11, outv), 0)[1],
            0)
        pltpu.sync_copy(outv, out_hbm.at[pl.ds(base * 6, CB * 6)])
        return carry

    lax.fori_loop(0, NCH, chunk_body, 0)


_mesh = plsc.VectorSubcoreMesh(core_axis_name="c", subcore_axis_name="s")

_lookup = functools.partial(
    pl.kernel,
    out_type=jax.ShapeDtypeStruct((B * C,), jnp.float32),
    mesh=_mesh,
    scratch_types=[
        pltpu.VMEM((CB * 3,), jnp.float32),
        pltpu.VMEM((CB,), jnp.float32),
        pltpu.VMEM((CB,), jnp.float32),
        pltpu.VMEM((CB,), jnp.int32),
        pltpu.VMEM((CB,), jnp.int32),
        pltpu.VMEM((CB,), jnp.int32),
        pltpu.VMEM((CB,), jnp.int32),
        pltpu.VMEM((CB, 8), jnp.float32),
        pltpu.VMEM((CB, 8), jnp.float32),
        pltpu.VMEM((CB, 8), jnp.float32),
        pltpu.VMEM((CB, 8), jnp.float32),
        pltpu.VMEM((CB * C,), jnp.float32),
        pltpu.SemaphoreType.DMA,
    ],
    compiler_params=pltpu.CompilerParams(
        needs_layout_passes=False, use_tc_tiling_on_sc=False),
)(_sc_lookup)


def kernel(inputs, texture, failv):
    xyz = inputs.reshape(-1)
    tex = jnp.transpose(texture, (0, 2, 3, 1))
    tex = jnp.pad(tex, ((0, 0), (0, 0), (0, 0), (0, 2)))
    table = tex.reshape(-1, 8)
    failrow = jnp.pad(failv, (0, 2)).reshape(1, 8)
    table = jnp.concatenate(
        [table, failrow, jnp.zeros((7, 8), jnp.float32)], axis=0)
    out = _lookup(xyz, table)
    return out.reshape(B, C)


# speedup vs baseline: 6.1763x; 6.1763x over previous
import functools

import jax
import jax.numpy as jnp
from jax import lax
from jax.experimental import pallas as pl
from jax.experimental.pallas import tpu as pltpu
from jax.experimental.pallas import tpu_sc as plsc

B = 1048576
RES = 512
C = 6
VFAIL = 6 * RES * RES
VROWS = VFAIL + 8

NC, NS = 2, 16
NW = NC * NS
RW = B // NW
CB = 1024
NCH = RW // CB
GB = 128
NG = CB // GB

NBLK = 6 * (RES // 8) * (RES // 128)
BPW = NBLK // NW


def _reformat_body(tex_hbm, failv_hbm, out_hbm, stag, obuf, fv, sem_i, sem_o):
    wid = lax.axis_index("s") * NC + lax.axis_index("c")
    iota = lax.iota(jnp.int32, 16)

    cpat = jnp.where(iota < 8, iota, iota - 8)
    spat = jnp.where(iota < 8, 0, 1)

    @pl.when(wid == 0)
    def _fail_row():
        pltpu.sync_copy(failv_hbm, fv)
        zero = jnp.zeros((16,), jnp.float32)
        for j in range(4):
            obuf[pl.ds(j * 16, 16)] = zero
        val = plsc.load_gather(fv, [jnp.minimum(iota, 7)])
        plsc.store_scatter(obuf, [iota], val, mask=iota < 8)
        pltpu.sync_copy(obuf.at[pl.ds(0, 64)],
                        out_hbm.at[pl.ds(VFAIL * 8, 64)])

    def blk_body(i, carry):
        blk = wid * BPW + i
        f = blk // 256
        rem = blk - f * 256
        tb = rem // 4
        sb = rem - tb * 4
        hs = []
        for c in range(C):
            hs.append(pltpu.async_copy(
                tex_hbm.at[f, c, pl.ds(tb * 8, 8), pl.ds(sb * 128, 128)],
                stag.at[c], sem_i))
        for h in hs:
            h.wait()

        def ivec(i2, carry2):
            t = i2 >> 6
            sg = i2 & 63
            val = plsc.load_gather(stag, [cpat, t + iota * 0, sg * 2 + spat])
            obuf[pl.ds(i2 * 16, 16)] = val
            return carry2

        lax.fori_loop(0, 512, ivec, 0)

        ho = []
        for t in range(8):
            off = f * 2097152 + (tb * 8 + t) * 4096 + sb * 1024
            ho.append(pltpu.async_copy(
                obuf.at[pl.ds(t * 1024, 1024)],
                out_hbm.at[pl.ds(off, 1024)], sem_o))
        for h in ho:
            h.wait()
        return carry

    lax.fori_loop(0, BPW, blk_body, 0)


_mesh = plsc.VectorSubcoreMesh(core_axis_name="c", subcore_axis_name="s")

_reformat = functools.partial(
    pl.kernel,
    out_type=jax.ShapeDtypeStruct((VROWS * 8,), jnp.float32),
    mesh=_mesh,
    scratch_types=[
        pltpu.VMEM((8, 8, 128), jnp.float32),
        pltpu.VMEM((8192,), jnp.float32),
        pltpu.VMEM((8,), jnp.float32),
        pltpu.SemaphoreType.DMA,
        pltpu.SemaphoreType.DMA,
    ],
    compiler_params=pltpu.CompilerParams(
        needs_layout_passes=False, use_tc_tiling_on_sc=True),
)(_reformat_body)


def _phase_a(g, xsv, ysv, zsv, wsv, wtv, i0, i1, i2, i3):
    off = g * 16
    sl = pl.ds(off, 16)
    x = xsv[sl]
    y = ysv[sl]
    z = zsv[sl]
    ax, ay, az = jnp.abs(x), jnp.abs(y), jnp.abs(z)
    is_x = (ax >= ay) & (ax >= az)
    is_y = (~is_x) & (ay >= az)
    ma = jnp.maximum(jnp.maximum(ax, ay), az)
    ms = jnp.maximum(ma, 1e-12)
    xpos, ypos, zpos = x >= 0, y >= 0, z >= 0
    u = jnp.where(is_x, jnp.where(xpos, -z, z),
                  jnp.where(is_y, x, jnp.where(zpos, x, -x))) / ms
    v = jnp.where(is_x, -y, jnp.where(is_y, jnp.where(ypos, z, -z), -y)) / ms
    face = jnp.where(is_x, jnp.where(xpos, 0, 1),
                     jnp.where(is_y, jnp.where(ypos, 2, 3),
                               jnp.where(zpos, 4, 5))).astype(jnp.int32)
    s = (u * 0.5 + 0.5) * RES - 0.5
    t = (v * 0.5 + 0.5) * RES - 0.5

    def flr(s):
        si_t = s.astype(jnp.int32)
        sf_t = si_t.astype(jnp.float32)
        adj = sf_t > s
        si = jnp.where(adj, si_t - 1, si_t)
        sf = jnp.where(adj, sf_t - 1.0, sf_t)
        return si, s - sf

    s0i, ws = flr(s)
    t0i, wt = flr(t)
    s0 = jnp.clip(s0i, 0, RES - 1)
    s1 = jnp.clip(s0i + 1, 0, RES - 1)
    t0 = jnp.clip(t0i, 0, RES - 1)
    t1 = jnp.clip(t0i + 1, 0, RES - 1)
    rb = face << 18
    r0 = rb + (t0 << 9)
    r1 = rb + (t1 << 9)
    fail = ma < 1e-12
    wsv[sl] = ws
    wtv[sl] = wt
    i0[sl] = jnp.where(fail, VFAIL, r0 + s0)
    i1[sl] = jnp.where(fail, VFAIL, r0 + s1)
    i2[sl] = jnp.where(fail, VFAIL, r1 + s0)
    i3[sl] = jnp.where(fail, VFAIL, r1 + s1)


def _phase_b(q, wsv, wtv, t00, t01, t10, t11, outs):
    off = q * 16
    iota = lax.iota(jnp.int32, 16)
    rowv = off + iota
    sl = pl.ds(off, 16)
    ws = wsv[sl]
    wt = wtv[sl]
    for ch in range(C):
        chv = jnp.full((16,), ch, jnp.int32)
        a00 = plsc.load_gather(t00, [rowv, chv])
        a01 = plsc.load_gather(t01, [rowv, chv])
        a10 = plsc.load_gather(t10, [rowv, chv])
        a11 = plsc.load_gather(t11, [rowv, chv])
        a = a00 + ws * (a01 - a00)
        b = a10 + ws * (a11 - a10)
        o = a + wt * (b - a)
        outs[ch][sl] = o


def _sc_lookup(xs_hbm, ys_hbm, zs_hbm, table_hbm,
               o0_hbm, o1_hbm, o2_hbm, o3_hbm, o4_hbm, o5_hbm,
               xsv, ysv, zsv, wsv, wtv, i0, i1, i2, i3,
               t00, t01, t10, t11,
               ov0, ov1, ov2, ov3, ov4, ov5, sem):
    outs_hbm = (o0_hbm, o1_hbm, o2_hbm, o3_hbm, o4_hbm, o5_hbm)
    outs_v = (ov0, ov1, ov2, ov3, ov4, ov5)
    wid = lax.axis_index("s") * NC + lax.axis_index("c")
    base0 = wid * RW

    def chunk_body(c, carry):
        base = base0 + c * CB
        pltpu.sync_copy(xs_hbm.at[pl.ds(base, CB)], xsv)
        pltpu.sync_copy(ys_hbm.at[pl.ds(base, CB)], ysv)
        pltpu.sync_copy(zs_hbm.at[pl.ds(base, CB)], zsv)
        lax.fori_loop(
            0, CB // 16,
            lambda g, _: (_phase_a(g, xsv, ysv, zsv, wsv, wtv,
                                   i0, i1, i2, i3), 0)[1],
            0)
        handles = []
        for iref, tref in ((i0, t00), (i1, t01), (i2, t10), (i3, t11)):
            for j in range(NG):
                handles.append(pltpu.async_copy(
                    table_hbm.at[iref.at[pl.ds(j * GB, GB)]],
                    tref.at[pl.ds(j * GB, GB)], sem))
        for h in handles:
            h.wait()
        lax.fori_loop(
            0, CB // 16,
            lambda q, _: (_phase_b(q, wsv, wtv, t00, t01, t10, t11, outs_v),
                          0)[1],
            0)
        for ch in range(C):
            pltpu.sync_copy(outs_v[ch], outs_hbm[ch].at[pl.ds(base, CB)])
        return carry

    lax.fori_loop(0, NCH, chunk_body, 0)


_lookup = functools.partial(
    pl.kernel,
    out_type=tuple(jax.ShapeDtypeStruct((B,), jnp.float32)
                   for _ in range(C)),
    mesh=_mesh,
    scratch_types=[
        pltpu.VMEM((CB,), jnp.float32),
        pltpu.VMEM((CB,), jnp.float32),
        pltpu.VMEM((CB,), jnp.float32),
        pltpu.VMEM((CB,), jnp.float32),
        pltpu.VMEM((CB,), jnp.float32),
        pltpu.VMEM((CB,), jnp.int32),
        pltpu.VMEM((CB,), jnp.int32),
        pltpu.VMEM((CB,), jnp.int32),
        pltpu.VMEM((CB,), jnp.int32),
        pltpu.VMEM((CB, 8), jnp.float32),
        pltpu.VMEM((CB, 8), jnp.float32),
        pltpu.VMEM((CB, 8), jnp.float32),
        pltpu.VMEM((CB, 8), jnp.float32),
        pltpu.VMEM((CB,), jnp.float32),
        pltpu.VMEM((CB,), jnp.float32),
        pltpu.VMEM((CB,), jnp.float32),
        pltpu.VMEM((CB,), jnp.float32),
        pltpu.VMEM((CB,), jnp.float32),
        pltpu.VMEM((CB,), jnp.float32),
        pltpu.SemaphoreType.DMA,
    ],
    compiler_params=pltpu.CompilerParams(
        needs_layout_passes=False, use_tc_tiling_on_sc=False),
)(_sc_lookup)


def kernel(inputs, texture, failv):
    xs = inputs[:, 0]
    ys = inputs[:, 1]
    zs = inputs[:, 2]
    failv8 = jnp.pad(failv, (0, 2))
    tableflat = _reformat(texture, failv8)
    table = tableflat.reshape(VROWS, 8)
    planes = _lookup(xs, ys, zs, table)
    return jnp.stack(planes, axis=1)


# speedup vs baseline: 8.3703x; 1.3552x over previous
import functools

import jax
import jax.numpy as jnp
from jax import lax
from jax.experimental import pallas as pl
from jax.experimental.pallas import tpu as pltpu
from jax.experimental.pallas import tpu_sc as plsc

B = 1048576
RES = 512
C = 6
VFAIL = 6 * RES * RES
VROWS = VFAIL + 8

NC, NS = 2, 16
NW = NC * NS
RW = B // NW
CB = 1024
NCH = RW // CB
GB = 128
NG = CB // GB

NBLK = 6 * (RES // 8) * (RES // 128)
BPW = NBLK // NW


def _reformat_body(tex_hbm, failv_hbm, out_hbm,
                   stag0, stag1, obuf0, obuf1, fv,
                   sem_i0, sem_i1, sem_o0, sem_o1):
    wid = lax.axis_index("s") * NC + lax.axis_index("c")
    iota = lax.iota(jnp.int32, 16)

    cpat = jnp.where(iota < 8, iota, iota - 8)
    spat = jnp.where(iota < 8, 0, 1)
    sets = ((stag0, obuf0, sem_i0, sem_o0),
            (stag1, obuf1, sem_i1, sem_o1))

    @pl.when(wid == 0)
    def _fail_row():
        pltpu.sync_copy(failv_hbm, fv)
        zero = jnp.zeros((16,), jnp.float32)
        for j in range(4):
            obuf0[pl.ds(j * 16, 16)] = zero
        val = plsc.load_gather(fv, [jnp.minimum(iota, 7)])
        plsc.store_scatter(obuf0, [iota], val, mask=iota < 8)
        pltpu.sync_copy(obuf0.at[pl.ds(0, 64)],
                        out_hbm.at[pl.ds(VFAIL * 8, 64)])

    def _coords(i):
        blk = wid * BPW + i
        f = blk // 256
        rem = blk - f * 256
        tb = rem // 4
        sb = rem - tb * 4
        return f, tb, sb

    def fire_in(i, st):
        stag, _, sem_i, _ = st
        f, tb, sb = _coords(i)
        for c in range(C):
            pltpu.async_copy(
                tex_hbm.at[f, c, pl.ds(tb * 8, 8), pl.ds(sb * 128, 128)],
                stag.at[c], sem_i)

    def drain_in(i, st):
        stag, _, sem_i, _ = st
        f, tb, sb = _coords(i)
        for c in range(C):
            pltpu.make_async_copy(
                tex_hbm.at[f, c, pl.ds(tb * 8, 8), pl.ds(sb * 128, 128)],
                stag.at[c], sem_i).wait()

    def fire_out(i, st):
        _, obuf, _, sem_o = st
        f, tb, sb = _coords(i)
        for t in range(8):
            off = f * 2097152 + (tb * 8 + t) * 4096 + sb * 1024
            pltpu.async_copy(obuf.at[pl.ds(t * 1024, 1024)],
                             out_hbm.at[pl.ds(off, 1024)], sem_o)

    def drain_out(st):
        _, obuf, _, sem_o = st
        for t in range(8):
            pltpu.make_async_copy(obuf.at[pl.ds(t * 1024, 1024)],
                                  out_hbm.at[pl.ds(t * 1024, 1024)],
                                  sem_o).wait()

    def proc(i, st, first):
        stag, obuf, _, _ = st
        drain_in(i, st)

        @pl.when(jnp.logical_not(first))
        def _():
            drain_out(st)

        def ivec(i4, carry2):
            for k in range(4):
                i2 = i4 * 4 + k
                t = i2 >> 6
                sg = i2 & 63
                val = plsc.load_gather(
                    stag, [cpat, t + iota * 0, sg * 2 + spat])
                obuf[pl.ds(i2 * 16, 16)] = val
            return carry2

        lax.fori_loop(0, 128, ivec, 0)
        fire_out(i, st)

    fire_in(0, sets[0])

    def pair_body(i2, carry):
        b = i2 * 2
        fire_in(b + 1, sets[1])
        proc(b, sets[0], b < 2)

        @pl.when(b + 2 < BPW)
        def _():
            fire_in(b + 2, sets[0])

        proc(b + 1, sets[1], b < 2)
        return carry

    lax.fori_loop(0, BPW // 2, pair_body, 0)
    drain_out(sets[0])
    drain_out(sets[1])


_mesh = plsc.VectorSubcoreMesh(core_axis_name="c", subcore_axis_name="s")

_reformat = functools.partial(
    pl.kernel,
    out_type=jax.ShapeDtypeStruct((VROWS * 8,), jnp.float32),
    mesh=_mesh,
    scratch_types=[
        pltpu.VMEM((8, 8, 128), jnp.float32),
        pltpu.VMEM((8, 8, 128), jnp.float32),
        pltpu.VMEM((8192,), jnp.float32),
        pltpu.VMEM((8192,), jnp.float32),
        pltpu.VMEM((8,), jnp.float32),
        pltpu.SemaphoreType.DMA,
        pltpu.SemaphoreType.DMA,
        pltpu.SemaphoreType.DMA,
        pltpu.SemaphoreType.DMA,
    ],
    compiler_params=pltpu.CompilerParams(
        needs_layout_passes=False, use_tc_tiling_on_sc=True),
)(_reformat_body)


def _phase_a(g, xsv, ysv, zsv, wsv, wtv, i0, i1, i2, i3):
    off = g * 16
    sl = pl.ds(off, 16)
    x = xsv[sl]
    y = ysv[sl]
    z = zsv[sl]
    ax, ay, az = jnp.abs(x), jnp.abs(y), jnp.abs(z)
    is_x = (ax >= ay) & (ax >= az)
    is_y = (~is_x) & (ay >= az)
    ma = jnp.maximum(jnp.maximum(ax, ay), az)
    ms = jnp.maximum(ma, 1e-12)
    xpos, ypos, zpos = x >= 0, y >= 0, z >= 0
    u = jnp.where(is_x, jnp.where(xpos, -z, z),
                  jnp.where(is_y, x, jnp.where(zpos, x, -x))) / ms
    v = jnp.where(is_x, -y, jnp.where(is_y, jnp.where(ypos, z, -z), -y)) / ms
    face = jnp.where(is_x, jnp.where(xpos, 0, 1),
                     jnp.where(is_y, jnp.where(ypos, 2, 3),
                               jnp.where(zpos, 4, 5))).astype(jnp.int32)
    s = (u * 0.5 + 0.5) * RES - 0.5
    t = (v * 0.5 + 0.5) * RES - 0.5

    def flr(s):
        si_t = s.astype(jnp.int32)
        sf_t = si_t.astype(jnp.float32)
        adj = sf_t > s
        si = jnp.where(adj, si_t - 1, si_t)
        sf = jnp.where(adj, sf_t - 1.0, sf_t)
        return si, s - sf

    s0i, ws = flr(s)
    t0i, wt = flr(t)
    s0 = jnp.clip(s0i, 0, RES - 1)
    s1 = jnp.clip(s0i + 1, 0, RES - 1)
    t0 = jnp.clip(t0i, 0, RES - 1)
    t1 = jnp.clip(t0i + 1, 0, RES - 1)
    rb = face << 18
    r0 = rb + (t0 << 9)
    r1 = rb + (t1 << 9)
    fail = ma < 1e-12
    wsv[sl] = ws
    wtv[sl] = wt
    i0[sl] = jnp.where(fail, VFAIL, r0 + s0)
    i1[sl] = jnp.where(fail, VFAIL, r0 + s1)
    i2[sl] = jnp.where(fail, VFAIL, r1 + s0)
    i3[sl] = jnp.where(fail, VFAIL, r1 + s1)


def _phase_b(q, wsv, wtv, t00, t01, t10, t11, outs):
    off = q * 16
    iota = lax.iota(jnp.int32, 16)
    rowv = off + iota
    sl = pl.ds(off, 16)
    ws = wsv[sl]
    wt = wtv[sl]
    for ch in range(C):
        chv = jnp.full((16,), ch, jnp.int32)
        a00 = plsc.load_gather(t00, [rowv, chv])
        a01 = plsc.load_gather(t01, [rowv, chv])
        a10 = plsc.load_gather(t10, [rowv, chv])
        a11 = plsc.load_gather(t11, [rowv, chv])
        a = a00 + ws * (a01 - a00)
        b = a10 + ws * (a11 - a10)
        o = a + wt * (b - a)
        outs[ch][sl] = o


def _sc_lookup(xs_hbm, ys_hbm, zs_hbm, table_hbm,
               o0_hbm, o1_hbm, o2_hbm, o3_hbm, o4_hbm, o5_hbm,
               xsv, ysv, zsv,
               wsv0, wtv0, i00, i01, i02, i03,
               wsv1, wtv1, i10, i11, i12, i13,
               t000, t001, t010, t011,
               t100, t101, t110, t111,
               ov0, ov1, ov2, ov3, ov4, ov5, sem0, sem1):
    outs_hbm = (o0_hbm, o1_hbm, o2_hbm, o3_hbm, o4_hbm, o5_hbm)
    outs_v = (ov0, ov1, ov2, ov3, ov4, ov5)
    sets = (
        (wsv0, wtv0, (i00, i01, i02, i03), (t000, t001, t010, t011), sem0),
        (wsv1, wtv1, (i10, i11, i12, i13), (t100, t101, t110, t111), sem1),
    )
    wid = lax.axis_index("s") * NC + lax.axis_index("c")
    base0 = wid * RW

    def stage_a(c, st):
        wsv, wtv, idxs, _, _ = st
        base = base0 + c * CB
        pltpu.sync_copy(xs_hbm.at[pl.ds(base, CB)], xsv)
        pltpu.sync_copy(ys_hbm.at[pl.ds(base, CB)], ysv)
        pltpu.sync_copy(zs_hbm.at[pl.ds(base, CB)], zsv)
        lax.fori_loop(
            0, CB // 16,
            lambda g, _: (_phase_a(g, xsv, ysv, zsv, wsv, wtv, *idxs), 0)[1],
            0)
        _fire(st)

    def _fire(st):
        _, _, idxs, tbufs, sem = st
        for iref, tref in zip(idxs, tbufs):
            for j in range(NG):
                pltpu.async_copy(
                    table_hbm.at[iref.at[pl.ds(j * GB, GB)]],
                    tref.at[pl.ds(j * GB, GB)], sem)

    def _drain(st):
        _, _, idxs, tbufs, sem = st
        for iref, tref in zip(idxs, tbufs):
            for j in range(NG):
                pltpu.make_async_copy(
                    table_hbm.at[iref.at[pl.ds(j * GB, GB)]],
                    tref.at[pl.ds(j * GB, GB)], sem).wait()

    def stage_b(c, st):
        wsv, wtv, _, tbufs, _ = st
        _drain(st)
        lax.fori_loop(
            0, CB // 16,
            lambda q, _: (_phase_b(q, wsv, wtv, *tbufs, outs_v), 0)[1],
            0)
        base = base0 + c * CB
        for ch in range(C):
            pltpu.sync_copy(outs_v[ch], outs_hbm[ch].at[pl.ds(base, CB)])

    stage_a(0, sets[0])

    def pair_body(i2, carry):
        c = i2 * 2
        stage_a(c + 1, sets[1])
        stage_b(c, sets[0])

        @pl.when(c + 2 < NCH)
        def _prefetch():
            stage_a(c + 2, sets[0])

        stage_b(c + 1, sets[1])
        return carry

    lax.fori_loop(0, NCH // 2, pair_body, 0)


_lookup = functools.partial(
    pl.kernel,
    out_type=tuple(jax.ShapeDtypeStruct((B,), jnp.float32)
                   for _ in range(C)),
    mesh=_mesh,
    scratch_types=[
        pltpu.VMEM((CB,), jnp.float32),
        pltpu.VMEM((CB,), jnp.float32),
        pltpu.VMEM((CB,), jnp.float32),
        pltpu.VMEM((CB,), jnp.float32),
        pltpu.VMEM((CB,), jnp.float32),
        pltpu.VMEM((CB,), jnp.int32),
        pltpu.VMEM((CB,), jnp.int32),
        pltpu.VMEM((CB,), jnp.int32),
        pltpu.VMEM((CB,), jnp.int32),
        pltpu.VMEM((CB,), jnp.float32),
        pltpu.VMEM((CB,), jnp.float32),
        pltpu.VMEM((CB,), jnp.int32),
        pltpu.VMEM((CB,), jnp.int32),
        pltpu.VMEM((CB,), jnp.int32),
        pltpu.VMEM((CB,), jnp.int32),
        pltpu.VMEM((CB, 8), jnp.float32),
        pltpu.VMEM((CB, 8), jnp.float32),
        pltpu.VMEM((CB, 8), jnp.float32),
        pltpu.VMEM((CB, 8), jnp.float32),
        pltpu.VMEM((CB, 8), jnp.float32),
        pltpu.VMEM((CB, 8), jnp.float32),
        pltpu.VMEM((CB, 8), jnp.float32),
        pltpu.VMEM((CB, 8), jnp.float32),
        pltpu.VMEM((CB,), jnp.float32),
        pltpu.VMEM((CB,), jnp.float32),
        pltpu.VMEM((CB,), jnp.float32),
        pltpu.VMEM((CB,), jnp.float32),
        pltpu.VMEM((CB,), jnp.float32),
        pltpu.VMEM((CB,), jnp.float32),
        pltpu.SemaphoreType.DMA,
        pltpu.SemaphoreType.DMA,
    ],
    compiler_params=pltpu.CompilerParams(
        needs_layout_passes=False, use_tc_tiling_on_sc=False),
)(_sc_lookup)


def kernel(inputs, texture, failv):
    xs = inputs[:, 0]
    ys = inputs[:, 1]
    zs = inputs[:, 2]
    failv8 = jnp.pad(failv, (0, 2))
    tableflat = _reformat(texture, failv8)
    table = tableflat.reshape(VROWS, 8)
    planes = _lookup(xs, ys, zs, table)
    return jnp.stack(planes, axis=1)
